# baseline (speedup 1.0000x reference)
import jax
import jax.numpy as jnp
from jax.experimental import pallas as pl
from jax.experimental.pallas import tpu as pltpu


def _vq_body(z_ref, emb_ref, kiota_ref, zq_ref, idx_ref):
    n = pl.program_id(1)
    z = z_ref[0]
    emb = emb_ref[0]
    kiota = kiota_ref[...]
    K = emb.shape[0]
    HW = z.shape[1]

    embs = emb * (-2.0)
    e2 = jnp.sum(emb * emb, axis=1, keepdims=True)
    z2 = jnp.sum(z * z, axis=0, keepdims=True)
    cross2 = jax.lax.dot_general(
        embs, z, (((1,), (0,)), ((), ())),
        preferred_element_type=jnp.float32)
    dist = (e2 + z2) + cross2

    m = jnp.min(dist, axis=0, keepdims=True)
    idxf = jnp.min(jnp.where(dist == m, kiota, float(K)),
                   axis=0, keepdims=True)
    onehot = (kiota == idxf).astype(jnp.float32)
    zq = jax.lax.dot_general(
        emb, onehot, (((0,), (0,)), ((), ())),
        preferred_element_type=jnp.float32)

    zq_ref[0] = zq
    idx_ref[0, 0, :] = idxf[0].astype(jnp.int32)


def kernel(feather, codebooks):
    N, C, H, W = feather.shape
    G, K, gs = codebooks.shape
    HW = H * W
    fr = feather.reshape(N, C, HW)
    kiota = jax.lax.broadcasted_iota(jnp.float32, (K, HW), 0)

    zq_r, idx_r = pl.pallas_call(
        _vq_body,
        grid=(G, N),
        in_specs=[
            pl.BlockSpec((1, gs, HW), lambda g, n: (n, g, 0)),
            pl.BlockSpec((1, K, gs), lambda g, n: (g, 0, 0)),
            pl.BlockSpec((K, HW), lambda g, n: (0, 0)),
        ],
        out_specs=[
            pl.BlockSpec((1, gs, HW), lambda g, n: (n, g, 0)),
            pl.BlockSpec((1, 1, HW), lambda g, n: (g * N + n, 0, 0)),
        ],
        out_shape=[
            jax.ShapeDtypeStruct((N, C, HW), jnp.float32),
            jax.ShapeDtypeStruct((G * N, 1, HW), jnp.int32),
        ],
        compiler_params=pltpu.CompilerParams(
            dimension_semantics=("parallel", "arbitrary")),
    )(fr, codebooks, kiota)

    zq = zq_r.reshape(N, C, H, W)
    code_index = idx_r.reshape(G, N, H, W).transpose(1, 0, 2, 3)
    return (zq, zq, code_index)


# speedup vs baseline: 1.0041x; 1.0041x over previous
import jax
import jax.numpy as jnp
from jax.experimental import pallas as pl
from jax.experimental.pallas import tpu as pltpu


def _vq_step(z_ref, emb_ref, zq_ref, idx_ref):
    z = z_ref[0]
    emb = emb_ref[0]
    K = emb.shape[0]
    HW = z.shape[1]

    embs = emb * (-2.0)
    e2 = jnp.sum(emb * emb, axis=1, keepdims=True)
    z2 = jnp.sum(z * z, axis=0, keepdims=True)
    cross2 = jax.lax.dot_general(
        embs, z, (((1,), (0,)), ((), ())),
        preferred_element_type=jnp.float32)
    dist = (e2 + z2) + cross2

    m = jnp.min(dist, axis=0, keepdims=True)
    kiota = jax.lax.broadcasted_iota(jnp.int32, (K, HW), 0)
    idx = jnp.min(jnp.where(dist == m, kiota, K), axis=0,
                  keepdims=True)
    onehot = (kiota == idx).astype(jnp.float32)
    zq = jax.lax.dot_general(
        emb, onehot, (((0,), (0,)), ((), ())),
        preferred_element_type=jnp.float32)

    zq_ref[0] = zq
    idx_ref[0, 0, :] = idx[0]


def kernel(feather, codebooks):
    N, C, H, W = feather.shape
    G, K, gs = codebooks.shape
    HW = H * W
    fr = feather.reshape(N, C, HW)

    def outer(fr_hbm, cb_hbm, zq_hbm, idx_hbm):
        pipeline = pltpu.emit_pipeline(
            _vq_step,
            grid=(G, N),
            in_specs=[
                pl.BlockSpec((1, gs, HW), lambda g, n: (n, g, 0)),
                pl.BlockSpec((1, K, gs), lambda g, n: (g, 0, 0)),
            ],
            out_specs=[
                pl.BlockSpec((1, gs, HW), lambda g, n: (n, g, 0)),
                pl.BlockSpec((1, 1, HW), lambda g, n: (g * N + n, 0, 0)),
            ],
        )
        pipeline(fr_hbm, cb_hbm, zq_hbm, idx_hbm)

    zq_r, idx_r = pl.pallas_call(
        outer,
        in_specs=[
            pl.BlockSpec(memory_space=pl.ANY),
            pl.BlockSpec(memory_space=pl.ANY),
        ],
        out_specs=[
            pl.BlockSpec(memory_space=pl.ANY),
            pl.BlockSpec(memory_space=pl.ANY),
        ],
        out_shape=[
            jax.ShapeDtypeStruct((N, C, HW), jnp.float32),
            jax.ShapeDtypeStruct((G * N, 1, HW), jnp.int32),
        ],
    )(fr, codebooks)

    zq = zq_r.reshape(N, C, H, W)
    code_index = idx_r.reshape(G, N, H, W).transpose(1, 0, 2, 3)
    return (zq, zq, code_index)
